# baseline (speedup 1.0000x reference)
import functools
import math

import jax
import jax.numpy as jnp
from jax import lax
from jax.experimental import pallas as pl
from jax.experimental.pallas import tpu as pltpu
from jax.experimental.pallas import tpu_sc as plsc

_B = 16384
_L = 50
_LP = 52
_D = 64
_C = 3
_NC = 2
_NS = 16
_NW = _NC * _NS
_BAGS_W = _B // _NW
_BAGS_CHUNK = 2
_IDX_CHUNK = _BAGS_CHUNK * _LP
_CHUNKS = _BAGS_W // _BAGS_CHUNK
_NBUF = 8
_SCALE = math.sqrt(_D) / _L
_UNROLL = 25


def _sc_body(texts_hbm, table_hbm, w_hbm, b_hbm, out_hbm,
             idx_all, w_v, b_v, out_v, *ring):
    rows = ring[:_NBUF]
    gsem = ring[_NBUF:]
    wid = lax.axis_index("s") * _NC + lax.axis_index("c")
    bag_base = wid * _BAGS_W
    idx_base = bag_base * _LP

    pltpu.sync_copy(texts_hbm.at[pl.ds(idx_base, _BAGS_W * _LP)], idx_all)
    pltpu.sync_copy(w_hbm, w_v)
    pltpu.sync_copy(b_hbm, b_v)

    wv = tuple(tuple(w_v[c, pl.ds(k * 16, 16)] * _SCALE for k in range(4))
               for c in range(_C))
    bvec = b_v[pl.ds(0, 16)]
    lane = jnp.arange(16, dtype=jnp.int32)
    lane_ok = lane < _C

    def gather(slot, c):
        return pltpu.make_async_copy(
            table_hbm.at[idx_all.at[pl.ds(c * _IDX_CHUNK, _IDX_CHUNK)]],
            rows[slot], gsem[slot])

    for s in range(_NBUF):
        gather(s, s).start()

    def outer(i, carry):
        for s in range(_NBUF):
            c = i * _NBUF + s
            gather(s, c).wait()
            for bb in range(_BAGS_CHUNK):
                def row_body(j, acc):
                    for u in range(_UNROLL):
                        r = bb * _LP + j * _UNROLL + u
                        acc = tuple(acc[k] + rows[s][r, pl.ds(k * 16, 16)]
                                    for k in range(4))
                    return acc
                acc = lax.fori_loop(
                    0, _L // _UNROLL, row_body,
                    tuple(jnp.zeros((16,), jnp.float32) for _ in range(4)))
                bag = c * _BAGS_CHUNK + bb
                sums = []
                for cls in range(_C):
                    t = acc[0] * wv[cls][0]
                    for k in range(1, 4):
                        t = t + acc[k] * wv[cls][k]
                    sums.append(jnp.sum(t))
                outvec = jnp.where(
                    lane == 0, sums[0],
                    jnp.where(lane == 1, sums[1],
                              jnp.where(lane == 2, sums[2], 0.0))) + bvec
                plsc.store_scatter(out_v, [bag * _C + lane], outvec,
                                   mask=lane_ok)

            @pl.when(c + _NBUF < _CHUNKS)
            def _():
                gather(s, c + _NBUF).start()
        return carry

    lax.fori_loop(0, _CHUNKS // _NBUF, outer, 0)
    pltpu.sync_copy(out_v, out_hbm.at[pl.ds(bag_base * _C, _BAGS_W * _C)])


@jax.jit
def _run(texts_flat, table, w, b16):
    mesh = plsc.VectorSubcoreMesh(core_axis_name="c", subcore_axis_name="s")
    return pl.kernel(
        _sc_body,
        out_type=jax.ShapeDtypeStruct((_B * _C,), jnp.float32),
        mesh=mesh,
        scratch_types=(
            [pltpu.VMEM((_BAGS_W * _LP,), jnp.int32),
             pltpu.VMEM((_C, _D), jnp.float32),
             pltpu.VMEM((16,), jnp.float32),
             pltpu.VMEM((_BAGS_W * _C,), jnp.float32)]
            + [pltpu.VMEM((_IDX_CHUNK, _D), jnp.float32)] * _NBUF
            + [pltpu.SemaphoreType.DMA] * _NBUF
        ),
        compiler_params=pltpu.CompilerParams(
            needs_layout_passes=False, use_tc_tiling_on_sc=False),
    )(texts_flat, table, w, b16)


def kernel(texts, table, W, b):
    texts_p = jnp.pad(texts, ((0, 0), (0, _LP - _L)))
    texts_flat = texts_p.reshape(-1)
    b16 = jnp.zeros((16,), b.dtype).at[:_C].set(b)
    return _run(texts_flat, table, W, b16).reshape(_B, _C)


# speedup vs baseline: 1.6453x; 1.6453x over previous
import functools
import math

import jax
import jax.numpy as jnp
from jax import lax
from jax.experimental import pallas as pl
from jax.experimental.pallas import tpu as pltpu
from jax.experimental.pallas import tpu_sc as plsc

_B = 16384
_L = 50
_LP = 52
_D = 64
_C = 3
_CP = 16
_V = 1000000
_NC = 2
_NS = 16
_NW = _NC * _NS
_BAGS_W = _B // _NW
_BAGS_CHUNK = 2
_IDX_CHUNK = _BAGS_CHUNK * _LP
_CHUNKS = _BAGS_W // _BAGS_CHUNK
_NBUF = 8
_SCALE = math.sqrt(_D) / _L
_UNROLL = 25
_NBLK = 8192
_NSTEP = -(-_V // _NBLK)


def _proj_body(t_ref, w_ref, out_ref):
    out_ref[...] = lax.dot_general(
        t_ref[...], w_ref[...],
        dimension_numbers=(((0,), (1,)), ((), ())),
        preferred_element_type=jnp.float32)


def _sc_body(texts_hbm, p_hbm, b_hbm, out_hbm,
             idx_all, b_v, out_v, *ring):
    rows = ring[:_NBUF]
    gsem = ring[_NBUF:]
    wid = lax.axis_index("s") * _NC + lax.axis_index("c")
    bag_base = wid * _BAGS_W
    idx_base = bag_base * _LP

    pltpu.sync_copy(texts_hbm.at[pl.ds(idx_base, _BAGS_W * _LP)], idx_all)
    pltpu.sync_copy(b_hbm, b_v)
    bvec = b_v[pl.ds(0, 16)]
    lane = jnp.arange(16, dtype=jnp.int32)
    lane_ok = lane < _C

    def gather(slot, c):
        return pltpu.make_async_copy(
            p_hbm.at[idx_all.at[pl.ds(c * _IDX_CHUNK, _IDX_CHUNK)]],
            rows[slot], gsem[slot])

    for s in range(_NBUF):
        gather(s, s).start()

    def outer(i, carry):
        for s in range(_NBUF):
            c = i * _NBUF + s
            gather(s, c).wait()
            for bb in range(_BAGS_CHUNK):
                def row_body(j, acc):
                    for u in range(_UNROLL):
                        r = bb * _LP + j * _UNROLL + u
                        acc = acc + rows[s][r, pl.ds(0, 16)]
                    return acc
                acc = lax.fori_loop(0, _L // _UNROLL, row_body,
                                    jnp.zeros((16,), jnp.float32))
                bag = c * _BAGS_CHUNK + bb
                plsc.store_scatter(out_v, [bag * _C + lane], acc + bvec,
                                   mask=lane_ok)

            @pl.when(c + _NBUF < _CHUNKS)
            def _():
                gather(s, c + _NBUF).start()
        return carry

    lax.fori_loop(0, _CHUNKS // _NBUF, outer, 0)
    pltpu.sync_copy(out_v, out_hbm.at[pl.ds(bag_base * _C, _BAGS_W * _C)])


@jax.jit
def _run(texts_flat, table_t, wp, b16):
    p = pl.pallas_call(
        _proj_body,
        grid=(_NSTEP,),
        in_specs=[
            pl.BlockSpec((_D, _NBLK), lambda j: (0, j)),
            pl.BlockSpec((_CP, _D), lambda j: (0, 0)),
        ],
        out_specs=pl.BlockSpec((_NBLK, _CP), lambda j: (j, 0)),
        out_shape=jax.ShapeDtypeStruct((_V, _CP), jnp.float32),
    )(table_t, wp)

    mesh = plsc.VectorSubcoreMesh(core_axis_name="c", subcore_axis_name="s")
    out = pl.kernel(
        _sc_body,
        out_type=jax.ShapeDtypeStruct((_B * _C,), jnp.float32),
        mesh=mesh,
        scratch_types=(
            [pltpu.VMEM((_BAGS_W * _LP,), jnp.int32),
             pltpu.VMEM((16,), jnp.float32),
             pltpu.VMEM((_BAGS_W * _C,), jnp.float32)]
            + [pltpu.VMEM((_IDX_CHUNK, _CP), jnp.float32)] * _NBUF
            + [pltpu.SemaphoreType.DMA] * _NBUF
        ),
        compiler_params=pltpu.CompilerParams(
            needs_layout_passes=False, use_tc_tiling_on_sc=False),
    )(texts_flat, p, b16)
    return out


def kernel(texts, table, W, b):
    texts_p = jnp.pad(texts, ((0, 0), (0, _LP - _L)))
    texts_flat = texts_p.reshape(-1)
    wp = jnp.zeros((_CP, _D), W.dtype).at[:_C].set(W * _SCALE)
    b16 = jnp.zeros((16,), b.dtype).at[:_C].set(b)
    return _run(texts_flat, table.T, wp, b16).reshape(_B, _C)


# speedup vs baseline: 1.8898x; 1.1486x over previous
import functools
import math

import jax
import jax.numpy as jnp
from jax import lax
from jax.experimental import pallas as pl
from jax.experimental.pallas import tpu as pltpu
from jax.experimental.pallas import tpu_sc as plsc

_B = 16384
_L = 50
_LP = 56
_D = 64
_C = 3
_V = 1000000
_NC = 2
_NS = 16
_NW = _NC * _NS
_BAGS_W = _B // _NW
_BAGS_CHUNK = 2
_IDX_CHUNK = _BAGS_CHUNK * _LP
_CHUNKS = _BAGS_W // _BAGS_CHUNK
_NBUF = 8
_SCALE = math.sqrt(_D) / _L
_NBLK = 8192
_NSTEP = -(-_V // _NBLK)


def _proj_body(t_ref, w_ref, o0_ref, o1_ref, o2_ref):
    res = lax.dot_general(
        w_ref[...], t_ref[...],
        dimension_numbers=(((1,), (0,)), ((), ())),
        preferred_element_type=jnp.float32)
    o0_ref[...] = res[0]
    o1_ref[...] = res[1]
    o2_ref[...] = res[2]


def _sc_body(texts_hbm, p0_hbm, p1_hbm, p2_hbm, b_hbm, out_hbm,
             idx_all, b_v, out_v, *ring):
    rows = [ring[3 * s:3 * s + 3] for s in range(_NBUF)]
    gsem = ring[3 * _NBUF:]
    p_hbm = (p0_hbm, p1_hbm, p2_hbm)
    wid = lax.axis_index("s") * _NC + lax.axis_index("c")
    bag_base = wid * _BAGS_W
    idx_base = bag_base * _LP

    pltpu.sync_copy(texts_hbm.at[pl.ds(idx_base, _BAGS_W * _LP)], idx_all)
    pltpu.sync_copy(b_hbm, b_v)
    bvec = b_v[pl.ds(0, 16)]
    lane = jnp.arange(16, dtype=jnp.int32)
    lane_ok = lane < _C
    tail_ok = (lane < 2).astype(jnp.float32)

    def gather(slot, c, cls):
        return pltpu.make_async_copy(
            p_hbm[cls].at[idx_all.at[pl.ds(c * _IDX_CHUNK, _IDX_CHUNK)]],
            rows[slot][cls].at[pl.ds(0, _IDX_CHUNK)], gsem[slot])

    for s in range(_NBUF):
        for cls in range(_C):
            gather(s, s, cls).start()

    def outer(i, carry):
        for s in range(_NBUF):
            c = i * _NBUF + s
            for cls in range(_C):
                gather(s, c, cls).wait()
            for bb in range(_BAGS_CHUNK):
                o = bb * _LP
                sums = []
                for cls in range(_C):
                    r = rows[s][cls]
                    t = (r[pl.ds(o, 16)] + r[pl.ds(o + 16, 16)]
                         + r[pl.ds(o + 32, 16)]
                         + r[pl.ds(o + 48, 16)] * tail_ok)
                    sums.append(jnp.sum(t))
                outvec = jnp.where(
                    lane == 0, sums[0],
                    jnp.where(lane == 1, sums[1],
                              jnp.where(lane == 2, sums[2], 0.0))) + bvec
                bag = c * _BAGS_CHUNK + bb
                plsc.store_scatter(out_v, [bag * _C + lane], outvec,
                                   mask=lane_ok)

            @pl.when(c + _NBUF < _CHUNKS)
            def _():
                for cls in range(_C):
                    gather(s, c + _NBUF, cls).start()
        return carry

    lax.fori_loop(0, _CHUNKS // _NBUF, outer, 0)
    pltpu.sync_copy(out_v, out_hbm.at[pl.ds(bag_base * _C, _BAGS_W * _C)])


@jax.jit
def _run(texts_flat, table_t, wp, b16):
    p0, p1, p2 = pl.pallas_call(
        _proj_body,
        grid=(_NSTEP,),
        in_specs=[
            pl.BlockSpec((_D, _NBLK), lambda j: (0, j)),
            pl.BlockSpec((8, _D), lambda j: (0, 0)),
        ],
        out_specs=[pl.BlockSpec((_NBLK,), lambda j: (j,))] * _C,
        out_shape=[jax.ShapeDtypeStruct((_V,), jnp.float32)] * _C,
    )(table_t, wp)

    mesh = plsc.VectorSubcoreMesh(core_axis_name="c", subcore_axis_name="s")
    out = pl.kernel(
        _sc_body,
        out_type=jax.ShapeDtypeStruct((_B * _C,), jnp.float32),
        mesh=mesh,
        scratch_types=(
            [pltpu.VMEM((_BAGS_W * _LP,), jnp.int32),
             pltpu.VMEM((16,), jnp.float32),
             pltpu.VMEM((_BAGS_W * _C,), jnp.float32)]
            + [pltpu.VMEM((_IDX_CHUNK + 16,), jnp.float32)] * (_C * _NBUF)
            + [pltpu.SemaphoreType.DMA] * _NBUF
        ),
        compiler_params=pltpu.CompilerParams(
            needs_layout_passes=False, use_tc_tiling_on_sc=False),
    )(texts_flat, p0, p1, p2, b16)
    return out


def kernel(texts, table, W, b):
    texts_p = jnp.pad(texts, ((0, 0), (0, _LP - _L)))
    texts_flat = texts_p.reshape(-1)
    wp = jnp.zeros((8, _D), W.dtype).at[:_C].set(W * _SCALE)
    b16 = jnp.zeros((16,), b.dtype).at[:_C].set(b)
    return _run(texts_flat, table.T, wp, b16).reshape(_B, _C)


# speedup vs baseline: 3.4616x; 1.8317x over previous
import functools
import math

import jax
import jax.numpy as jnp
from jax import lax
from jax.experimental import pallas as pl
from jax.experimental.pallas import tpu as pltpu
from jax.experimental.pallas import tpu_sc as plsc

_B = 16384
_L = 50
_LP = 52
_D = 64
_C = 3
_V = 1000000
_VP = 1 << 20
_NC = 2
_NS = 16
_NW = _NC * _NS
_BAGS_W = _B // _NW
_BAGS_CHUNK = 2
_IDX_CHUNK = _BAGS_CHUNK * _LP
_CHUNKS = _BAGS_W // _BAGS_CHUNK
_NBUF = 8
_SCALE = math.sqrt(_D) / _L
_NBLK = 8192
_VW = _VP // _NW
_VCH = 2048
_NCH = _VW // _VCH


def _proj_body(t_ref, w_ref, o0_ref, o1_ref, o2_ref):
    res = lax.dot_general(
        w_ref[...], t_ref[...],
        dimension_numbers=(((1,), (0,)), ((), ())),
        preferred_element_type=jnp.float32)
    o0_ref[...] = res[0]
    o1_ref[...] = res[1]
    o2_ref[...] = res[2]


def _ilv_body(p0_hbm, p1_hbm, p2_hbm, out_hbm,
              in00, in01, in02, in10, in11, in12, blk0, blk1,
              isem0, isem1, osem0, osem1):
    ins = ((in00, in01, in02), (in10, in11, in12))
    blks = (blk0, blk1)
    isems = (isem0, isem1)
    osems = (osem0, osem1)
    p_hbm = (p0_hbm, p1_hbm, p2_hbm)
    wid = lax.axis_index("s") * _NC + lax.axis_index("c")
    base = wid * _VW
    lane = jnp.arange(16, dtype=jnp.int32)

    def in_start(k, s):
        for c in range(_C):
            pltpu.async_copy(p_hbm[c].at[pl.ds(base + k * _VCH, _VCH)],
                             ins[s][c], isems[s])

    def in_wait(s):
        for c in range(_C):
            pltpu.make_async_copy(p_hbm[c].at[pl.ds(0, _VCH)],
                                  ins[s][c], isems[s]).wait()

    def out_desc(k, s):
        return pltpu.make_async_copy(
            blks[s], out_hbm.at[pl.ds(base + k * _VCH, _VCH)], osems[s])

    in_start(0, 0)
    for k in range(_NCH):
        s = k % 2
        in_wait(s)
        if k + 1 < _NCH:
            in_start(k + 1, 1 - s)
        if k >= 2:
            out_desc(k - 2, s).wait()

        def scat(j, carry):
            row = j * 16 + lane
            for c in range(_C):
                plsc.store_scatter(blks[s], [row, jnp.full((16,), c, jnp.int32)],
                                   ins[s][c][pl.ds(j * 16, 16)])
            return carry

        lax.fori_loop(0, _VCH // 16, scat, 0)
        out_desc(k, s).start()
    out_desc(_NCH - 2, 0 if _NCH % 2 == 0 else 1).wait()
    out_desc(_NCH - 1, 1 if _NCH % 2 == 0 else 0).wait()


def _sc_body(texts_hbm, p16_hbm, b_hbm, out_hbm,
             idx_all, b_v, out_v, *ring):
    rows = ring[:_NBUF]
    gsem = ring[_NBUF:]
    wid = lax.axis_index("s") * _NC + lax.axis_index("c")
    bag_base = wid * _BAGS_W
    idx_base = bag_base * _LP

    pltpu.sync_copy(texts_hbm.at[pl.ds(idx_base, _BAGS_W * _LP)], idx_all)
    pltpu.sync_copy(b_hbm, b_v)
    bvec = b_v[pl.ds(0, 16)]
    lane = jnp.arange(16, dtype=jnp.int32)
    lane_ok = lane < _C

    def gather(slot, c):
        return pltpu.make_async_copy(
            p16_hbm.at[idx_all.at[pl.ds(c * _IDX_CHUNK, _IDX_CHUNK)]],
            rows[slot], gsem[slot])

    for s in range(_NBUF):
        gather(s, s).start()

    def outer(i, carry):
        for s in range(_NBUF):
            c = i * _NBUF + s
            gather(s, c).wait()
            for bb in range(_BAGS_CHUNK):
                def row_body(j, accs):
                    a0, a1 = accs
                    r = bb * _LP + j * 2
                    return (a0 + rows[s][r, pl.ds(0, 16)],
                            a1 + rows[s][r + 1, pl.ds(0, 16)])
                z = jnp.zeros((16,), jnp.float32)
                a0, a1 = lax.fori_loop(0, _L // 2, row_body, (z, z))
                bag = c * _BAGS_CHUNK + bb
                plsc.store_scatter(out_v, [bag * _C + lane],
                                   a0 + a1 + bvec, mask=lane_ok)

            @pl.when(c + _NBUF < _CHUNKS)
            def _():
                gather(s, c + _NBUF).start()
        return carry

    lax.fori_loop(0, _CHUNKS // _NBUF, outer, 0)
    pltpu.sync_copy(out_v, out_hbm.at[pl.ds(bag_base * _C, _BAGS_W * _C)])


@jax.jit
def _run(texts_flat, table_t, wp, b16):
    p0, p1, p2 = pl.pallas_call(
        _proj_body,
        grid=(-(-_V // _NBLK),),
        in_specs=[
            pl.BlockSpec((_D, _NBLK), lambda j: (0, j)),
            pl.BlockSpec((8, _D), lambda j: (0, 0)),
        ],
        out_specs=[pl.BlockSpec((_NBLK,), lambda j: (j,))] * _C,
        out_shape=[jax.ShapeDtypeStruct((_VP,), jnp.float32)] * _C,
    )(table_t, wp)

    mesh = plsc.VectorSubcoreMesh(core_axis_name="c", subcore_axis_name="s")
    sc_params = pltpu.CompilerParams(
        needs_layout_passes=False, use_tc_tiling_on_sc=False)

    p16 = pl.kernel(
        _ilv_body,
        out_type=jax.ShapeDtypeStruct((_VP, 16), jnp.float32),
        mesh=mesh,
        scratch_types=(
            [pltpu.VMEM((_VCH,), jnp.float32)] * 6
            + [pltpu.VMEM((_VCH, 16), jnp.float32)] * 2
            + [pltpu.SemaphoreType.DMA] * 4
        ),
        compiler_params=sc_params,
    )(p0, p1, p2)

    out = pl.kernel(
        _sc_body,
        out_type=jax.ShapeDtypeStruct((_B * _C,), jnp.float32),
        mesh=mesh,
        scratch_types=(
            [pltpu.VMEM((_BAGS_W * _LP,), jnp.int32),
             pltpu.VMEM((16,), jnp.float32),
             pltpu.VMEM((_BAGS_W * _C,), jnp.float32)]
            + [pltpu.VMEM((_IDX_CHUNK, 16), jnp.float32)] * _NBUF
            + [pltpu.SemaphoreType.DMA] * _NBUF
        ),
        compiler_params=sc_params,
    )(texts_flat, p16, b16)
    return out


def kernel(texts, table, W, b):
    texts_p = jnp.pad(texts, ((0, 0), (0, _LP - _L)))
    texts_flat = texts_p.reshape(-1)
    wp = jnp.zeros((8, _D), W.dtype).at[:_C].set(W * _SCALE)
    b16 = jnp.zeros((16,), b.dtype).at[:_C].set(b)
    return _run(texts_flat, table.T, wp, b16).reshape(_B, _C)
